# baseline (speedup 1.0000x reference)
import functools
import numpy as np
import jax
import jax.numpy as jnp
from jax import lax
from jax.experimental import pallas as pl
from jax.experimental.pallas import tpu as pltpu
from jax.experimental.pallas import tpu_sc as plsc

f32 = jnp.float32
bf16 = jnp.bfloat16
i32 = jnp.int32

N = 10000
E = 320000
G = 64
HD = 128
NB = 1000
EB = 2000
NW = 32
EPW = E // NW
CH = 80
NCH = EPW // CH
NSC = 10240
NPS = NSC // 16

_SDS = jax.ShapeDtypeStruct


def _silu(v):
    return v * (1.0 / (1.0 + jnp.exp(-v)))


def _dot(a, b):
    return jnp.dot(a, b, preferred_element_type=f32)


def _freq_expand(c3, F_ref):
    return (c3[:, 0:1] * F_ref[0:1, :] + c3[:, 1:2] * F_ref[1:2, :]
            + c3[:, 2:3] * F_ref[2:3, :])


def _k1_body(oh_ref, fc_ref, g_ref, Wnf_ref, bias0_ref, Fn_ref, WA_ref, WB_ref,
             gW_ref, W1a_ref, W1b_ref, latB_ref, x_ref, xs_ref, xd_ref):
    oh = oh_ref[...]
    x = _dot(oh, Wnf_ref[...]) + bias0_ref[...]
    ff = _freq_expand(fc_ref[...], Fn_ref[...])
    x = x + _dot(jnp.sin(ff), WA_ref[...]) + _dot(jnp.cos(ff), WB_ref[...])
    g = g_ref[...]
    ohg = (lax.broadcasted_iota(i32, (NB, G), 1) == g).astype(f32)
    x = x + _dot(ohg, gW_ref[...])
    x_ref[...] = x
    xs_ref[...] = (_dot(x, W1a_ref[...]) + _dot(ohg, latB_ref[...])).astype(bf16)
    xd_ref[...] = _dot(x, W1b_ref[...]).astype(bf16)


def _k3_body(x_ref, a0_ref, a1_ref, c0_ref, c1_ref, g_ref,
             nw1a_ref, nw1b_ref, nb1_ref, nw2_ref, nb2_ref,
             W1a_ref, W1b_ref, latB_ref, xn_ref, xs_ref, xd_ref):
    x = x_ref[...]
    agg = a0_ref[...] + a1_ref[...]
    cnt = c0_ref[...] + c1_ref[...]
    inv = 1.0 / jnp.maximum(cnt[:, 0:1], 1.0)
    am = agg * inv
    t1 = _silu(_dot(x, nw1a_ref[...]) + _dot(am, nw1b_ref[...]) + nb1_ref[...])
    o = _silu(_dot(t1, nw2_ref[...]) + nb2_ref[...])
    xn = x + o
    xn_ref[...] = xn
    ohg = (lax.broadcasted_iota(i32, (NB, G), 1) == g_ref[...]).astype(f32)
    xs_ref[...] = (_dot(xn, W1a_ref[...]) + _dot(ohg, latB_ref[...])).astype(bf16)
    xd_ref[...] = _dot(xn, W1b_ref[...]).astype(bf16)


def _k3h_body(x_ref, a0_ref, a1_ref, c0_ref, c1_ref,
              nw1a_ref, nw1b_ref, nb1_ref, nw2_ref, nb2_ref,
              Wc1_ref, bc1_ref, ac_ref, Wc2_ref, bc2_ref,
              Wt1_ref, bt1_ref, at_ref, Wt2_ref, bt2_ref,
              coord_ref, types_ref):
    x = x_ref[...]
    agg = a0_ref[...] + a1_ref[...]
    cnt = c0_ref[...] + c1_ref[...]
    inv = 1.0 / jnp.maximum(cnt[:, 0:1], 1.0)
    am = agg * inv
    t1 = _silu(_dot(x, nw1a_ref[...]) + _dot(am, nw1b_ref[...]) + nb1_ref[...])
    o = _silu(_dot(t1, nw2_ref[...]) + nb2_ref[...])
    h = x + o
    ac = ac_ref[0, 0]
    v = _dot(h, Wc1_ref[...]) + bc1_ref[...]
    v = jnp.maximum(v, 0.0) + ac * jnp.minimum(v, 0.0)
    coord_ref[...] = _dot(v, Wc2_ref[...]) + bc2_ref[...]
    at = at_ref[0, 0]
    w = _dot(h, Wt1_ref[...]) + bt1_ref[...]
    w = jnp.maximum(w, 0.0) + at * jnp.minimum(w, 0.0)
    types_ref[...] = _dot(w, Wt2_ref[...]) + bt2_ref[...]


def _k5_body(gs_ref, gd_ref, fcs_ref, fcd_ref, Fe_ref, WdS_ref, WdC_ref,
             W2_ref, b2_ref, e_ref):
    fd = fcd_ref[...][:, 0:3] - fcs_ref[...][:, 0:3]
    de = _freq_expand(fd, Fe_ref[...])
    pre = gs_ref[...].astype(f32) + gd_ref[...].astype(f32) \
        + _dot(jnp.sin(de), WdS_ref[...]) \
        + _dot(jnp.cos(de), WdC_ref[...])
    m = _silu(pre)
    e = _silu(_dot(m, W2_ref[...]) + b2_ref[...])
    e_ref[...] = e


def _full(shape):
    return pl.BlockSpec(shape, lambda i: tuple(0 for _ in shape))


def _blk(shape):
    def imap(i):
        return (i,) + tuple(0 for _ in shape[1:])
    return pl.BlockSpec(shape, imap)


def _sc_kernels():
    scmesh = plsc.VectorSubcoreMesh(core_axis_name="c", subcore_axis_name="s")
    _cp = pltpu.CompilerParams(use_tc_tiling_on_sc=False)
    k2 = functools.partial(
        pl.kernel,
        compiler_params=_cp,
        out_type=[_SDS((E, 16), f32), _SDS((E, 16), f32), _SDS((2, NSC, 16), f32)],
        mesh=scmesh,
        scratch_types=[
            pltpu.VMEM((NCH, CH), i32), pltpu.VMEM((NCH, CH), i32),
            pltpu.VMEM((CH, 16), f32), pltpu.VMEM((CH, 16), f32),
            pltpu.VMEM((CH, 16), f32),
            pltpu.VMEM_SHARED((NSC, 16), f32),
            pltpu.SemaphoreType.DMA, pltpu.SemaphoreType.DMA,
        ],
    )(_k2_setup_body)
    k4 = functools.partial(
        pl.kernel,
        compiler_params=_cp,
        out_type=[_SDS((E, HD), bf16), _SDS((E, HD), bf16)],
        mesh=scmesh,
        scratch_types=[
            pltpu.VMEM((NCH, CH), i32), pltpu.VMEM((NCH, CH), i32),
            pltpu.VMEM((CH, HD), bf16), pltpu.VMEM((CH, HD), bf16),
            pltpu.SemaphoreType.DMA, pltpu.SemaphoreType.DMA,
        ],
    )(_k4_gather_body)
    k6 = functools.partial(
        pl.kernel,
        compiler_params=_cp,
        out_type=_SDS((2, NSC, HD), f32),
        mesh=scmesh,
        scratch_types=[
            pltpu.VMEM((NCH, CH), i32),
            pltpu.VMEM((CH, HD), f32),
            pltpu.VMEM_SHARED((NSC, HD), f32),
            pltpu.SemaphoreType.DMA,
        ],
    )(_k6_scatter_body)
    return k2, k4, k6


def _k2_setup_body(src3_hbm, dst3_hbm, fc16_hbm, z16_hbm, ones_hbm,
                   fcs_out, fcd_out, cnt_out,
                   si_v, di_v, bufa, bufb, ones_v, cntsh, sema, semb):
    cid = lax.axis_index("c")
    sid = lax.axis_index("s")
    wid = sid * 2 + cid
    pltpu.sync_copy(z16_hbm, cntsh.at[pl.ds(sid * NPS, NPS)])
    pltpu.sync_copy(ones_hbm, ones_v)
    pltpu.sync_copy(src3_hbm.at[wid], si_v)
    pltpu.sync_copy(dst3_hbm.at[wid], di_v)
    plsc.subcore_barrier()

    def chunk(j, carry):
        off = wid * EPW + j * CH
        ca = pltpu.async_copy(fc16_hbm.at[si_v.at[j]], bufa, sema)
        cb = pltpu.async_copy(fc16_hbm.at[di_v.at[j]], bufb, semb)
        ca.wait()
        cb.wait()
        pltpu.sync_copy(bufa, fcs_out.at[pl.ds(off, CH)])
        pltpu.sync_copy(bufb, fcd_out.at[pl.ds(off, CH)])
        pltpu.sync_copy(ones_v, cntsh.at[si_v.at[j]], add=True)
        return carry

    lax.fori_loop(0, NCH, chunk, 0)
    plsc.subcore_barrier()
    pltpu.sync_copy(cntsh.at[pl.ds(sid * NPS, NPS)],
                    cnt_out.at[cid, pl.ds(sid * NPS, NPS)])


def _k4_gather_body(xs_hbm, xd_hbm, src3_hbm, dst3_hbm, gs_out, gd_out,
                    si_v, di_v, bufa, bufb, sema, semb):
    cid = lax.axis_index("c")
    sid = lax.axis_index("s")
    wid = sid * 2 + cid
    pltpu.sync_copy(src3_hbm.at[wid], si_v)
    pltpu.sync_copy(dst3_hbm.at[wid], di_v)

    def chunk(j, carry):
        off = wid * EPW + j * CH
        ca = pltpu.async_copy(xs_hbm.at[si_v.at[j]], bufa, sema)
        cb = pltpu.async_copy(xd_hbm.at[di_v.at[j]], bufb, semb)
        ca.wait()
        cb.wait()
        pltpu.sync_copy(bufa, gs_out.at[pl.ds(off, CH)])
        pltpu.sync_copy(bufb, gd_out.at[pl.ds(off, CH)])
        return carry

    lax.fori_loop(0, NCH, chunk, 0)


def _k6_scatter_body(e_hbm, src3_hbm, z128_hbm, agg_out, si_v, ebuf, aggsh, sem):
    cid = lax.axis_index("c")
    sid = lax.axis_index("s")
    wid = sid * 2 + cid
    pltpu.sync_copy(z128_hbm, aggsh.at[pl.ds(sid * NPS, NPS)])
    pltpu.sync_copy(src3_hbm.at[wid], si_v)
    plsc.subcore_barrier()

    def chunk(j, carry):
        off = wid * EPW + j * CH
        pltpu.sync_copy(e_hbm.at[pl.ds(off, CH)], ebuf)
        pltpu.sync_copy(ebuf, aggsh.at[si_v.at[j]], add=True)
        return carry

    lax.fori_loop(0, NCH, chunk, 0)
    plsc.subcore_barrier()
    pltpu.sync_copy(aggsh.at[pl.ds(sid * NPS, NPS)],
                    agg_out.at[cid, pl.ds(sid * NPS, NPS)])


def kernel(atom_onehot, frac_coords, lattices, t, z, edge_index, node2graph,
           W_node, b_node, W_latent, b_latent,
           edge_w1, edge_b1, edge_w2, edge_b2,
           node_w1, node_b1, node_w2, node_b2,
           Wc1, bc1, a_c, Wc2, bc2, Wt1, bt1, a_t, Wt2, bt2):
    src = edge_index[0].astype(i32)
    dst = edge_index[1].astype(i32)
    src3d = src.reshape(NW, NCH, CH)
    dst3d = dst.reshape(NW, NCH, CH)
    n2g = node2graph.astype(i32).reshape(N, 1)
    fc16 = jnp.pad(frac_coords.astype(f32), ((0, 0), (0, 13)))
    oh128 = jnp.pad(atom_onehot.astype(f32), ((0, 0), (0, 28)))

    half = 64
    emb = jnp.exp(jnp.arange(half, dtype=f32) * -(np.log(10000.0) / (half - 1)))
    te = t[:, None] * emb[None, :]
    t_emb = jnp.concatenate([jnp.sin(te), jnp.cos(te)], axis=-1)
    lat_ip = jnp.einsum('gij,gkj->gik', lattices, lattices).reshape(G, 9)

    Wla, Wlb = W_latent[0:64], W_latent[64:192]
    Wlc, Wld = W_latent[192:256], W_latent[256:316]
    Wnf = jnp.pad(W_node @ Wla, ((0, 28), (0, 0)))
    bias0 = (b_node @ Wla + b_latent).reshape(1, HD)
    gW = t_emb @ Wlb + z @ Wlc
    WldA, WldB = Wld[0:30], Wld[30:60]

    Fn = np.zeros((3, 30), np.float32)
    for k in range(10):
        for c in range(3):
            Fn[c, 3 * k + c] = (2.0 ** k) * 2.0 * np.pi
    Fn = jnp.asarray(Fn)
    Fe = np.zeros((3, 30), np.float32)
    for c in range(3):
        for k in range(10):
            Fe[c, 10 * c + k] = 2.0 * np.pi * k
    Fe = jnp.asarray(Fe)

    W1a = [edge_w1[l, 0:128] for l in range(4)]
    W1b = [edge_w1[l, 128:256] for l in range(4)]
    latB = [lat_ip @ edge_w1[l, 256:265] + edge_b1[l] for l in range(4)]
    WdS = [edge_w1[l, 265:295] for l in range(4)]
    WdC = [edge_w1[l, 295:325] for l in range(4)]
    W2 = [edge_w2[l] for l in range(4)]
    b2 = [edge_b2[l].reshape(1, HD) for l in range(4)]
    nw1a = [node_w1[l, 0:128] for l in range(4)]
    nw1b = [node_w1[l, 128:256] for l in range(4)]
    nb1 = [node_b1[l].reshape(1, HD) for l in range(4)]
    nw2 = [node_w2[l] for l in range(4)]
    nb2 = [node_b2[l].reshape(1, HD) for l in range(4)]

    z16 = jnp.zeros((NPS, 16), f32)
    z128 = jnp.zeros((NPS, HD), f32)
    ones16 = jnp.ones((CH, 16), f32)

    _k2_setup, _k4_gather, _k6_scatter = _sc_kernels()
    fcs, fcd, cnt2 = _k2_setup(src3d, dst3d, fc16, z16, ones16)
    c0, c1 = cnt2[0, :N], cnt2[1, :N]

    grid_n = N // NB
    x, xs, xd = pl.pallas_call(
        _k1_body,
        grid=(grid_n,),
        in_specs=[_blk((NB, 128)), _blk((NB, 3)), _blk((NB, 1)),
                  _full((128, HD)), _full((1, HD)), _full((3, 30)),
                  _full((30, HD)), _full((30, HD)), _full((G, HD)),
                  _full((HD, HD)), _full((HD, HD)), _full((G, HD))],
        out_specs=[_blk((NB, HD)), _blk((NB, HD)), _blk((NB, HD))],
        out_shape=[_SDS((N, HD), f32), _SDS((N, HD), bf16), _SDS((N, HD), bf16)],
    )(oh128, frac_coords.astype(f32), n2g, Wnf, bias0, Fn, WldA, WldB, gW,
      W1a[0], W1b[0], latB[0])

    grid_e = E // EB
    for l in range(4):
        gs, gd = _k4_gather(xs, xd, src3d, dst3d)
        e = pl.pallas_call(
            _k5_body,
            grid=(grid_e,),
            in_specs=[_blk((EB, HD)), _blk((EB, HD)),
                      _blk((EB, 16)), _blk((EB, 16)),
                      _full((3, 30)), _full((30, HD)), _full((30, HD)),
                      _full((HD, HD)), _full((1, HD))],
            out_specs=_blk((EB, HD)),
            out_shape=_SDS((E, HD), f32),
        )(gs, gd, fcs, fcd, Fe, WdS[l], WdC[l], W2[l], b2[l])
        agg2 = _k6_scatter(e, src3d, z128)
        a0, a1 = agg2[0, :N], agg2[1, :N]
        if l < 3:
            x, xs, xd = pl.pallas_call(
                _k3_body,
                grid=(grid_n,),
                in_specs=[_blk((NB, HD)), _blk((NB, HD)), _blk((NB, HD)),
                          _blk((NB, 16)), _blk((NB, 16)), _blk((NB, 1)),
                          _full((HD, HD)), _full((HD, HD)), _full((1, HD)),
                          _full((HD, HD)), _full((1, HD)),
                          _full((HD, HD)), _full((HD, HD)), _full((G, HD))],
                out_specs=[_blk((NB, HD))] * 3,
                out_shape=[_SDS((N, HD), f32), _SDS((N, HD), bf16),
                           _SDS((N, HD), bf16)],
            )(x, a0, a1, c0, c1, n2g, nw1a[l], nw1b[l], nb1[l], nw2[l], nb2[l],
              W1a[l + 1], W1b[l + 1], latB[l + 1])
        else:
            coord, types = pl.pallas_call(
                _k3h_body,
                grid=(grid_n,),
                in_specs=[_blk((NB, HD)), _blk((NB, HD)), _blk((NB, HD)),
                          _blk((NB, 16)), _blk((NB, 16)),
                          _full((HD, HD)), _full((HD, HD)), _full((1, HD)),
                          _full((HD, HD)), _full((1, HD)),
                          _full((HD, 64)), _full((1, 64)), _full((1, 1)),
                          _full((64, 6)), _full((1, 6)),
                          _full((HD, 64)), _full((1, 64)), _full((1, 1)),
                          _full((64, 100)), _full((1, 100))],
                out_specs=[_blk((NB, 6)), _blk((NB, 100))],
                out_shape=[_SDS((N, 6), f32), _SDS((N, 100), f32)],
            )(x, a0, a1, c0, c1, nw1a[l], nw1b[l], nb1[l], nw2[l], nb2[l],
              Wc1, bc1.reshape(1, 64), a_c.reshape(1, 1), Wc2, bc2.reshape(1, 6),
              Wt1, bt1.reshape(1, 64), a_t.reshape(1, 1), Wt2, bt2.reshape(1, 100))
    return coord, types


# speedup vs baseline: 1.6013x; 1.6013x over previous
import functools
import numpy as np
import jax
import jax.numpy as jnp
from jax import lax
from jax.experimental import pallas as pl
from jax.experimental.pallas import tpu as pltpu
from jax.experimental.pallas import tpu_sc as plsc

f32 = jnp.float32
i32 = jnp.int32

N = 10000
E = 320000
G = 64
HD = 128
NB = 1000
EB = 2000
NW = 32
EPW = E // NW
CH = 80
NCH = EPW // CH
EH = E // 2
EPWH = EH // NW
CHH = 40
NCHH = EPWH // CHH
NSC = 10240
NPS = NSC // 16

_SDS = jax.ShapeDtypeStruct


def _silu(v):
    return v * (1.0 / (1.0 + jnp.exp(-v)))


def _dot(a, b):
    return jnp.dot(a, b, preferred_element_type=f32)


def _freq_expand(c3, F_ref):
    return (c3[:, 0:1] * F_ref[0:1, :] + c3[:, 1:2] * F_ref[1:2, :]
            + c3[:, 2:3] * F_ref[2:3, :])


def _k1_body(oh_ref, fc_ref, g_ref, Wnf_ref, bias0_ref, Fn_ref, WA_ref, WB_ref,
             gW_ref, W1a_ref, W1b_ref, latB_ref, x_ref, xs_ref, xd_ref):
    oh = oh_ref[...]
    x = _dot(oh, Wnf_ref[...]) + bias0_ref[...]
    ff = _freq_expand(fc_ref[...], Fn_ref[...])
    x = x + _dot(jnp.sin(ff), WA_ref[...]) + _dot(jnp.cos(ff), WB_ref[...])
    g = g_ref[...]
    ohg = (lax.broadcasted_iota(i32, (NB, G), 1) == g).astype(f32)
    x = x + _dot(ohg, gW_ref[...])
    x_ref[...] = x
    xs_ref[...] = _dot(x, W1a_ref[...]) + _dot(ohg, latB_ref[...])
    xd_ref[...] = _dot(x, W1b_ref[...])


def _k3_body(x_ref, a0_ref, a1_ref, a2_ref, a3_ref, c0_ref, c1_ref, g_ref,
             nw1a_ref, nw1b_ref, nb1_ref, nw2_ref, nb2_ref,
             W1a_ref, W1b_ref, latB_ref, xn_ref, xs_ref, xd_ref):
    x = x_ref[...]
    agg = (a0_ref[...] + a1_ref[...]) + (a2_ref[...] + a3_ref[...])
    cnt = c0_ref[...] + c1_ref[...]
    inv = 1.0 / jnp.maximum(cnt[:, 0:1], 1.0)
    am = agg * inv
    t1 = _silu(_dot(x, nw1a_ref[...]) + _dot(am, nw1b_ref[...]) + nb1_ref[...])
    o = _silu(_dot(t1, nw2_ref[...]) + nb2_ref[...])
    xn = x + o
    xn_ref[...] = xn
    ohg = (lax.broadcasted_iota(i32, (NB, G), 1) == g_ref[...]).astype(f32)
    xs_ref[...] = _dot(xn, W1a_ref[...]) + _dot(ohg, latB_ref[...])
    xd_ref[...] = _dot(xn, W1b_ref[...])


def _k3h_body(x_ref, a0_ref, a1_ref, a2_ref, a3_ref, c0_ref, c1_ref,
              nw1a_ref, nw1b_ref, nb1_ref, nw2_ref, nb2_ref,
              Wc1_ref, bc1_ref, ac_ref, Wc2_ref, bc2_ref,
              Wt1_ref, bt1_ref, at_ref, Wt2_ref, bt2_ref,
              coord_ref, types_ref):
    x = x_ref[...]
    agg = (a0_ref[...] + a1_ref[...]) + (a2_ref[...] + a3_ref[...])
    cnt = c0_ref[...] + c1_ref[...]
    inv = 1.0 / jnp.maximum(cnt[:, 0:1], 1.0)
    am = agg * inv
    t1 = _silu(_dot(x, nw1a_ref[...]) + _dot(am, nw1b_ref[...]) + nb1_ref[...])
    o = _silu(_dot(t1, nw2_ref[...]) + nb2_ref[...])
    h = x + o
    ac = ac_ref[0, 0]
    v = _dot(h, Wc1_ref[...]) + bc1_ref[...]
    v = jnp.maximum(v, 0.0) + ac * jnp.minimum(v, 0.0)
    coord_ref[...] = _dot(v, Wc2_ref[...]) + bc2_ref[...]
    at = at_ref[0, 0]
    w = _dot(h, Wt1_ref[...]) + bt1_ref[...]
    w = jnp.maximum(w, 0.0) + at * jnp.minimum(w, 0.0)
    types_ref[...] = _dot(w, Wt2_ref[...]) + bt2_ref[...]


def _k5_body(gs_ref, gd_ref, fcs_ref, fcd_ref, Fe_ref, WdS_ref, WdC_ref,
             W2_ref, b2_ref, e_ref):
    fd = fcd_ref[...][:, 0:3] - fcs_ref[...][:, 0:3]
    de = _freq_expand(fd, Fe_ref[...])
    pre = gs_ref[...] + gd_ref[...] + _dot(jnp.sin(de), WdS_ref[...]) \
        + _dot(jnp.cos(de), WdC_ref[...])
    m = _silu(pre)
    e = _silu(_dot(m, W2_ref[...]) + b2_ref[...])
    e_ref[...] = e


def _full(shape):
    return pl.BlockSpec(shape, lambda i: tuple(0 for _ in shape))


def _blk(shape, off=0):
    def imap(i):
        return (i + off,) + tuple(0 for _ in shape[1:])
    return pl.BlockSpec(shape, imap)


def _k2_setup_body(src3_hbm, dst3_hbm, fc16_hbm, z16_hbm, ones_hbm,
                   fcs_out, fcd_out, cnt_out,
                   si_v, di_v, bufa, bufb, ones_v, cntsh, sema, semb):
    cid = lax.axis_index("c")
    sid = lax.axis_index("s")
    wid = sid * 2 + cid
    pltpu.sync_copy(z16_hbm, cntsh.at[pl.ds(sid * NPS, NPS)])
    pltpu.sync_copy(ones_hbm, ones_v)
    pltpu.sync_copy(src3_hbm.at[wid], si_v)
    pltpu.sync_copy(dst3_hbm.at[wid], di_v)
    plsc.subcore_barrier()

    def chunk(j, carry):
        off = wid * EPW + j * CH
        ca = pltpu.async_copy(fc16_hbm.at[si_v.at[j]], bufa, sema)
        cb = pltpu.async_copy(fc16_hbm.at[di_v.at[j]], bufb, semb)
        ca.wait()
        cb.wait()
        pltpu.sync_copy(bufa, fcs_out.at[pl.ds(off, CH)])
        pltpu.sync_copy(bufb, fcd_out.at[pl.ds(off, CH)])
        pltpu.sync_copy(ones_v, cntsh.at[si_v.at[j]], add=True)
        return carry

    lax.fori_loop(0, NCH, chunk, 0)
    plsc.subcore_barrier()
    pltpu.sync_copy(cntsh.at[pl.ds(sid * NPS, NPS)],
                    cnt_out.at[cid, pl.ds(sid * NPS, NPS)])


def _k4_gather_body(xs_hbm, xd_hbm, src3_hbm, dst3_hbm, gs_out, gd_out,
                    si_v, di_v, bufa, bufb, sema, semb):
    cid = lax.axis_index("c")
    sid = lax.axis_index("s")
    wid = sid * 2 + cid
    pltpu.sync_copy(src3_hbm.at[wid], si_v)
    pltpu.sync_copy(dst3_hbm.at[wid], di_v)

    def chunk(j, carry):
        off = wid * EPWH + j * CHH
        ca = pltpu.async_copy(xs_hbm.at[si_v.at[j]], bufa, sema)
        cb = pltpu.async_copy(xd_hbm.at[di_v.at[j]], bufb, semb)
        ca.wait()
        cb.wait()
        pltpu.sync_copy(bufa, gs_out.at[pl.ds(off, CHH)])
        pltpu.sync_copy(bufb, gd_out.at[pl.ds(off, CHH)])
        return carry

    lax.fori_loop(0, NCHH, chunk, 0)


def _k6_scatter_body(e_hbm, src3_hbm, z128_hbm, agg_out, si_v, ebuf, aggsh, sem):
    cid = lax.axis_index("c")
    sid = lax.axis_index("s")
    wid = sid * 2 + cid
    pltpu.sync_copy(z128_hbm, aggsh.at[pl.ds(sid * NPS, NPS)])
    pltpu.sync_copy(src3_hbm.at[wid], si_v)
    plsc.subcore_barrier()

    def chunk(j, carry):
        off = wid * EPWH + j * CHH
        pltpu.sync_copy(e_hbm.at[pl.ds(off, CHH)], ebuf)
        pltpu.sync_copy(ebuf, aggsh.at[si_v.at[j]], add=True)
        return carry

    lax.fori_loop(0, NCHH, chunk, 0)
    plsc.subcore_barrier()
    pltpu.sync_copy(aggsh.at[pl.ds(sid * NPS, NPS)],
                    agg_out.at[cid, pl.ds(sid * NPS, NPS)])


def _sc_kernels():
    scmesh = plsc.VectorSubcoreMesh(core_axis_name="c", subcore_axis_name="s")
    _cp = pltpu.CompilerParams(use_tc_tiling_on_sc=False)
    k2 = functools.partial(
        pl.kernel,
        compiler_params=_cp,
        out_type=[_SDS((E, 16), f32), _SDS((E, 16), f32),
                  _SDS((2, NSC, 16), f32)],
        mesh=scmesh,
        scratch_types=[
            pltpu.VMEM((NCH, CH), i32), pltpu.VMEM((NCH, CH), i32),
            pltpu.VMEM((CH, 16), f32), pltpu.VMEM((CH, 16), f32),
            pltpu.VMEM((CH, 16), f32),
            pltpu.VMEM_SHARED((NSC, 16), f32),
            pltpu.SemaphoreType.DMA, pltpu.SemaphoreType.DMA,
        ],
    )(_k2_setup_body)
    k4 = functools.partial(
        pl.kernel,
        compiler_params=_cp,
        out_type=[_SDS((EH, HD), f32), _SDS((EH, HD), f32)],
        mesh=scmesh,
        scratch_types=[
            pltpu.VMEM((NCHH, CHH), i32), pltpu.VMEM((NCHH, CHH), i32),
            pltpu.VMEM((CHH, HD), f32), pltpu.VMEM((CHH, HD), f32),
            pltpu.SemaphoreType.DMA, pltpu.SemaphoreType.DMA,
        ],
    )(_k4_gather_body)
    k6 = functools.partial(
        pl.kernel,
        compiler_params=_cp,
        out_type=_SDS((2, NSC, HD), f32),
        mesh=scmesh,
        scratch_types=[
            pltpu.VMEM((NCHH, CHH), i32),
            pltpu.VMEM((CHH, HD), f32),
            pltpu.VMEM_SHARED((NSC, HD), f32),
            pltpu.SemaphoreType.DMA,
        ],
    )(_k6_scatter_body)
    return k2, k4, k6


def kernel(atom_onehot, frac_coords, lattices, t, z, edge_index, node2graph,
           W_node, b_node, W_latent, b_latent,
           edge_w1, edge_b1, edge_w2, edge_b2,
           node_w1, node_b1, node_w2, node_b2,
           Wc1, bc1, a_c, Wc2, bc2, Wt1, bt1, a_t, Wt2, bt2):
    src = edge_index[0].astype(i32)
    dst = edge_index[1].astype(i32)
    src3d = src.reshape(NW, NCH, CH)
    dst3d = dst.reshape(NW, NCH, CH)
    srch = [src[:EH].reshape(NW, NCHH, CHH), src[EH:].reshape(NW, NCHH, CHH)]
    dsth = [dst[:EH].reshape(NW, NCHH, CHH), dst[EH:].reshape(NW, NCHH, CHH)]
    n2g = node2graph.astype(i32).reshape(N, 1)
    fc16 = jnp.pad(frac_coords.astype(f32), ((0, 0), (0, 13)))
    oh128 = jnp.pad(atom_onehot.astype(f32), ((0, 0), (0, 28)))

    half = 64
    emb = jnp.exp(jnp.arange(half, dtype=f32) * -(np.log(10000.0) / (half - 1)))
    te = t[:, None] * emb[None, :]
    t_emb = jnp.concatenate([jnp.sin(te), jnp.cos(te)], axis=-1)
    lat_ip = jnp.einsum('gij,gkj->gik', lattices, lattices).reshape(G, 9)

    Wla, Wlb = W_latent[0:64], W_latent[64:192]
    Wlc, Wld = W_latent[192:256], W_latent[256:316]
    Wnf = jnp.pad(W_node @ Wla, ((0, 28), (0, 0)))
    bias0 = (b_node @ Wla + b_latent).reshape(1, HD)
    gW = t_emb @ Wlb + z @ Wlc
    WldA, WldB = Wld[0:30], Wld[30:60]

    Fn = np.zeros((3, 30), np.float32)
    for k in range(10):
        for c in range(3):
            Fn[c, 3 * k + c] = (2.0 ** k) * 2.0 * np.pi
    Fn = jnp.asarray(Fn)
    Fe = np.zeros((3, 30), np.float32)
    for c in range(3):
        for k in range(10):
            Fe[c, 10 * c + k] = 2.0 * np.pi * k
    Fe = jnp.asarray(Fe)

    W1a = [edge_w1[l, 0:128] for l in range(4)]
    W1b = [edge_w1[l, 128:256] for l in range(4)]
    latB = [lat_ip @ edge_w1[l, 256:265] + edge_b1[l] for l in range(4)]
    WdS = [edge_w1[l, 265:295] for l in range(4)]
    WdC = [edge_w1[l, 295:325] for l in range(4)]
    W2 = [edge_w2[l] for l in range(4)]
    b2 = [edge_b2[l].reshape(1, HD) for l in range(4)]
    nw1a = [node_w1[l, 0:128] for l in range(4)]
    nw1b = [node_w1[l, 128:256] for l in range(4)]
    nb1 = [node_b1[l].reshape(1, HD) for l in range(4)]
    nw2 = [node_w2[l] for l in range(4)]
    nb2 = [node_b2[l].reshape(1, HD) for l in range(4)]

    z16 = jnp.zeros((NPS, 16), f32)
    z128 = jnp.zeros((NPS, HD), f32)
    ones16 = jnp.ones((CH, 16), f32)

    _k2_setup, _k4_gather, _k6_scatter = _sc_kernels()
    fcs, fcd, cnt2 = _k2_setup(src3d, dst3d, fc16, z16, ones16)
    c0, c1 = cnt2[0, :N], cnt2[1, :N]

    grid_n = N // NB
    x, xs, xd = pl.pallas_call(
        _k1_body,
        grid=(grid_n,),
        in_specs=[_blk((NB, 128)), _blk((NB, 3)), _blk((NB, 1)),
                  _full((128, HD)), _full((1, HD)), _full((3, 30)),
                  _full((30, HD)), _full((30, HD)), _full((G, HD)),
                  _full((HD, HD)), _full((HD, HD)), _full((G, HD))],
        out_specs=[_blk((NB, HD)), _blk((NB, HD)), _blk((NB, HD))],
        out_shape=[_SDS((N, HD), f32)] * 3,
    )(oh128, frac_coords.astype(f32), n2g, Wnf, bias0, Fn, WldA, WldB, gW,
      W1a[0], W1b[0], latB[0])

    grid_h = EH // EB
    for l in range(4):
        aggs = []
        for h in range(2):
            gs, gd = _k4_gather(xs, xd, srch[h], dsth[h])
            e = pl.pallas_call(
                _k5_body,
                grid=(grid_h,),
                in_specs=[_blk((EB, HD)), _blk((EB, HD)),
                          _blk((EB, 16), off=h * grid_h),
                          _blk((EB, 16), off=h * grid_h),
                          _full((3, 30)), _full((30, HD)), _full((30, HD)),
                          _full((HD, HD)), _full((1, HD))],
                out_specs=_blk((EB, HD)),
                out_shape=_SDS((EH, HD), f32),
            )(gs, gd, fcs, fcd, Fe, WdS[l], WdC[l], W2[l], b2[l])
            agg2 = _k6_scatter(e, srch[h], z128)
            aggs += [agg2[0, :N], agg2[1, :N]]
        if l < 3:
            x, xs, xd = pl.pallas_call(
                _k3_body,
                grid=(grid_n,),
                in_specs=[_blk((NB, HD))] * 5 +
                         [_blk((NB, 16)), _blk((NB, 16)), _blk((NB, 1)),
                          _full((HD, HD)), _full((HD, HD)), _full((1, HD)),
                          _full((HD, HD)), _full((1, HD)),
                          _full((HD, HD)), _full((HD, HD)), _full((G, HD))],
                out_specs=[_blk((NB, HD))] * 3,
                out_shape=[_SDS((N, HD), f32)] * 3,
            )(x, aggs[0], aggs[1], aggs[2], aggs[3], c0, c1, n2g,
              nw1a[l], nw1b[l], nb1[l], nw2[l], nb2[l],
              W1a[l + 1], W1b[l + 1], latB[l + 1])
        else:
            coord, types = pl.pallas_call(
                _k3h_body,
                grid=(grid_n,),
                in_specs=[_blk((NB, HD))] * 5 +
                         [_blk((NB, 16)), _blk((NB, 16)),
                          _full((HD, HD)), _full((HD, HD)), _full((1, HD)),
                          _full((HD, HD)), _full((1, HD)),
                          _full((HD, 64)), _full((1, 64)), _full((1, 1)),
                          _full((64, 6)), _full((1, 6)),
                          _full((HD, 64)), _full((1, 64)), _full((1, 1)),
                          _full((64, 100)), _full((1, 100))],
                out_specs=[_blk((NB, 6)), _blk((NB, 100))],
                out_shape=[_SDS((N, 6), f32), _SDS((N, 100), f32)],
            )(x, aggs[0], aggs[1], aggs[2], aggs[3], c0, c1,
              nw1a[l], nw1b[l], nb1[l], nw2[l], nb2[l],
              Wc1, bc1.reshape(1, 64), a_c.reshape(1, 1), Wc2,
              bc2.reshape(1, 6),
              Wt1, bt1.reshape(1, 64), a_t.reshape(1, 1), Wt2,
              bt2.reshape(1, 100))
    return coord, types
